# baseline (speedup 1.0000x reference)
import functools

import jax
import jax.numpy as jnp
from jax import lax
from jax.experimental import pallas as pl
from jax.experimental.pallas import tpu as pltpu
from jax.experimental.pallas import tpu_sc as plsc

N = 10000
E = 320000
D_FEAT = 128
HIDDEN = 64
NUM_CLASSES = 40
NUM_LAYERS = 8

NC = 2
NS = 16
HC = HIDDEN // NC
NP = N
CHUNK = 128
EPT = 20480
NCHUNK = EPT // CHUNK
RPT = NP // NS
RSTEP = 125


def _mm_relu_body(x_ref, w_ref, b_ref, o_ref):
    acc = jnp.dot(x_ref[...], w_ref[...], preferred_element_type=jnp.float32)
    o_ref[...] = jnp.maximum(acc + b_ref[...], 0.0)


def _mm_relu(x, w, b):
    m_blk = 2000
    grid = (N // m_blk,)
    return pl.pallas_call(
        _mm_relu_body,
        grid=grid,
        in_specs=[
            pl.BlockSpec((m_blk, D_FEAT), lambda i: (i, 0)),
            pl.BlockSpec((D_FEAT, HIDDEN), lambda i: (0, 0)),
            pl.BlockSpec((1, HIDDEN), lambda i: (0, 0)),
        ],
        out_specs=pl.BlockSpec((m_blk, HIDDEN), lambda i: (i, 0)),
        out_shape=jax.ShapeDtypeStruct((N, HIDDEN), jnp.float32),
    )(x, w, b.reshape(1, HIDDEN))


def _head_body(p_ref, w_ref, b_ref, o_ref):
    logits = jnp.dot(p_ref[...], w_ref[...], preferred_element_type=jnp.float32)
    logits = logits + b_ref[...]
    mx = jnp.max(logits, axis=-1, keepdims=True)
    z = logits - mx
    lse = jnp.log(jnp.sum(jnp.exp(z), axis=-1, keepdims=True))
    o_ref[...] = z - lse


def _head(pooled, w, b):
    m_blk = 2000
    grid = (N // m_blk,)
    return pl.pallas_call(
        _head_body,
        grid=grid,
        in_specs=[
            pl.BlockSpec((m_blk, HIDDEN), lambda i: (i, 0)),
            pl.BlockSpec((HIDDEN, NUM_CLASSES), lambda i: (0, 0)),
            pl.BlockSpec((1, NUM_CLASSES), lambda i: (0, 0)),
        ],
        out_specs=pl.BlockSpec((m_blk, NUM_CLASSES), lambda i: (i, 0)),
        out_shape=jax.ShapeDtypeStruct((N, NUM_CLASSES), jnp.float32),
    )(pooled, w, b.reshape(1, NUM_CLASSES))


def _sc_body(h0_hbm, src_hbm, dst_hbm, w_hbm, pai_hbm, out_hbm,
             h_acc, hp_a, hp_b, src_v, dst_v, w_v, rows_va, rows_vb,
             pooled_v, zstage_v, coef_v, gsem_a, gsem_b, ssem_a, ssem_b):
    cid = lax.axis_index("c")
    sid = lax.axis_index("s")
    row0 = sid * RPT

    pltpu.sync_copy(src_hbm.at[sid], src_v)
    pltpu.sync_copy(dst_hbm.at[sid], dst_v)
    pltpu.sync_copy(w_hbm.at[sid], w_v)

    pltpu.sync_copy(pai_hbm, coef_v)
    coefs = jnp.exp(coef_v[...])
    c0 = coefs[0]

    off = cid * NP

    def _adj(i, _):
        for g in range(CHUNK // 16):
            sl = pl.ds(g * 16, 16)
            src_v[i, sl] = src_v[i, sl] + off
        return 0
    lax.fori_loop(0, NCHUNK, _adj, 0)

    def _zrow(r, _):
        for v in range(HC // 16):
            zstage_v[r, pl.ds(v * 16, 16)] = jnp.zeros((16,), jnp.float32)
        return 0
    lax.fori_loop(0, RSTEP, _zrow, 0)

    for j in range(RPT // RSTEP):
        r0 = row0 + j * RSTEP
        stage = rows_va.at[pl.ds(0, RSTEP)]
        pltpu.sync_copy(h0_hbm.at[pl.ds(off + r0, RSTEP)], stage)

        def _pinit(r, _, j=j):
            for v in range(HC // 16):
                sl = pl.ds(v * 16, 16)
                pooled_v[j * RSTEP + r, sl] = c0 * rows_va[r, sl]
            return 0
        lax.fori_loop(0, RSTEP, _pinit, 0)
        pltpu.sync_copy(zstage_v, h_acc.at[pl.ds(r0, RSTEP)])
    plsc.subcore_barrier()

    def _scale(buf, k):
        def _group(g, _):
            wvec = w_v[k, pl.ds(g * 16, 16)]
            for e16 in range(16):
                e = g * 16 + e16
                w = wvec[e16]
                for v in range(HC // 16):
                    sl = pl.ds(v * 16, 16)
                    buf[e, sl] = buf[e, sl] * w
            return 0
        lax.fori_loop(0, CHUNK // 16, _group, 0)

    for l in range(NUM_LAYERS):
        h_in = h0_hbm if l == 0 else (hp_a if l % 2 == 1 else hp_b)
        h_nx = hp_a if l % 2 == 0 else hp_b

        def _pair(k2, _):
            k0 = k2 * 2
            k1 = k0 + 1
            ga = pltpu.async_copy(h_in.at[src_v.at[k0]], rows_va, gsem_a)
            gb = pltpu.async_copy(h_in.at[src_v.at[k1]], rows_vb, gsem_b)
            ga.wait()
            _scale(rows_va, k0)
            sa = pltpu.async_copy(rows_va, h_acc.at[dst_v.at[k0]], ssem_a,
                                  add=True)
            gb.wait()
            _scale(rows_vb, k1)
            sb = pltpu.async_copy(rows_vb, h_acc.at[dst_v.at[k1]], ssem_b,
                                  add=True)
            sa.wait()
            sb.wait()
            return 0
        lax.fori_loop(0, NCHUNK // 2, _pair, 0)
        plsc.subcore_barrier()

        cl = coefs[l + 1]
        for j in range(RPT // RSTEP):
            r0 = row0 + j * RSTEP
            stage = rows_va.at[pl.ds(0, RSTEP)]
            pltpu.sync_copy(h_acc.at[pl.ds(r0, RSTEP)], stage)
            pltpu.sync_copy(stage, h_nx.at[pl.ds(off + r0, RSTEP)])

            def _pacc(r, _, j=j):
                for v in range(HC // 16):
                    sl = pl.ds(v * 16, 16)
                    pooled_v[j * RSTEP + r, sl] = (
                        pooled_v[j * RSTEP + r, sl] + cl * rows_va[r, sl])
                return 0
            lax.fori_loop(0, RSTEP, _pacc, 0)
            if l < NUM_LAYERS - 1:
                pltpu.sync_copy(zstage_v, h_acc.at[pl.ds(r0, RSTEP)])
        plsc.subcore_barrier()

    pltpu.sync_copy(pooled_v, out_hbm.at[cid, pl.ds(row0, RPT)])


@functools.partial(
    pl.kernel,
    out_type=jax.ShapeDtypeStruct((NC, NP, HC), jnp.float32),
    mesh=plsc.VectorSubcoreMesh(core_axis_name="c", subcore_axis_name="s",
                                num_cores=NC, num_subcores=NS),
    scratch_types=[
        pltpu.VMEM_SHARED((NP, HC), jnp.float32),
        pltpu.HBM((NC * NP, HC), jnp.float32),
        pltpu.HBM((NC * NP, HC), jnp.float32),
        pltpu.VMEM((NCHUNK, CHUNK), jnp.int32),
        pltpu.VMEM((NCHUNK, CHUNK), jnp.int32),
        pltpu.VMEM((NCHUNK, CHUNK), jnp.float32),
        pltpu.VMEM((CHUNK, HC), jnp.float32),
        pltpu.VMEM((CHUNK, HC), jnp.float32),
        pltpu.VMEM((RSTEP, HC), jnp.float32),
        pltpu.VMEM((RPT, HC), jnp.float32),
        pltpu.VMEM((16,), jnp.float32),
        pltpu.SemaphoreType.DMA,
        pltpu.SemaphoreType.DMA,
        pltpu.SemaphoreType.DMA,
        pltpu.SemaphoreType.DMA,
    ],
    compiler_params=pltpu.CompilerParams(use_tc_tiling_on_sc=False),
)
def _sc_propagate(h0_hbm, src_hbm, dst_hbm, w_hbm, pai_hbm, out_hbm,
                  h_acc, hp_a, hp_b, src_v, dst_v, w_v, rows_va, rows_vb,
                  zstage_v, pooled_v, coef_v, gsem_a, gsem_b, ssem_a, ssem_b):
    _sc_body(h0_hbm, src_hbm, dst_hbm, w_hbm, pai_hbm, out_hbm,
             h_acc, hp_a, hp_b, src_v, dst_v, w_v, rows_va, rows_vb,
             pooled_v, zstage_v, coef_v, gsem_a, gsem_b, ssem_a, ssem_b)


def _pad_edges(a):
    a = a.reshape(NS, E // NS)
    a = jnp.pad(a, ((0, 0), (0, EPT - E // NS)))
    return a.reshape(NS, NCHUNK, CHUNK)


def kernel(x, edge_index, edge_weight, mask, type, W0, b0, W1, b1, pai, convW):
    del mask, type, convW
    src = _pad_edges(edge_index[0].astype(jnp.int32))
    dst = _pad_edges(edge_index[1].astype(jnp.int32))
    w = _pad_edges(edge_weight.astype(jnp.float32))
    pai_pad = jnp.pad(pai.reshape(-1), (0, 16 - (NUM_LAYERS + 1)))

    h0 = _mm_relu(x, W0, b0)
    h0_split = h0.reshape(NP, NC, HC).transpose(1, 0, 2).reshape(NC * NP, HC)
    pooled_split = _sc_propagate(h0_split, src, dst, w, pai_pad)
    pooled = pooled_split.transpose(1, 0, 2).reshape(NP, HIDDEN)
    return _head(pooled, W1, b1)


# speedup vs baseline: 1.1381x; 1.1381x over previous
import functools

import jax
import jax.numpy as jnp
from jax import lax
from jax.experimental import pallas as pl
from jax.experimental.pallas import tpu as pltpu
from jax.experimental.pallas import tpu_sc as plsc

N = 10000
E = 320000
D_FEAT = 128
HIDDEN = 64
NUM_CLASSES = 40
NUM_LAYERS = 8

NC = 2
NS = 16
HC = HIDDEN // NC
NP = N
CHUNK = 128
EPT = 20480
NCHUNK = EPT // CHUNK
RPT = NP // NS
RSTEP = 125


def _mm_relu_body(x_ref, w_ref, b_ref, o_ref):
    acc = jnp.dot(x_ref[...], w_ref[...], preferred_element_type=jnp.float32)
    o_ref[...] = jnp.maximum(acc + b_ref[...], 0.0)


def _mm_relu(x, w, b):
    m_blk = 2000
    grid = (N // m_blk,)
    return pl.pallas_call(
        _mm_relu_body,
        grid=grid,
        in_specs=[
            pl.BlockSpec((m_blk, D_FEAT), lambda i: (i, 0)),
            pl.BlockSpec((D_FEAT, HIDDEN), lambda i: (0, 0)),
            pl.BlockSpec((1, HIDDEN), lambda i: (0, 0)),
        ],
        out_specs=pl.BlockSpec((m_blk, HIDDEN), lambda i: (i, 0)),
        out_shape=jax.ShapeDtypeStruct((N, HIDDEN), jnp.float32),
    )(x, w, b.reshape(1, HIDDEN))


def _head_body(p_ref, w_ref, b_ref, o_ref):
    logits = jnp.dot(p_ref[...], w_ref[...], preferred_element_type=jnp.float32)
    logits = logits + b_ref[...]
    mx = jnp.max(logits, axis=-1, keepdims=True)
    z = logits - mx
    lse = jnp.log(jnp.sum(jnp.exp(z), axis=-1, keepdims=True))
    o_ref[...] = z - lse


def _head(pooled, w, b):
    m_blk = 2000
    grid = (N // m_blk,)
    return pl.pallas_call(
        _head_body,
        grid=grid,
        in_specs=[
            pl.BlockSpec((m_blk, HIDDEN), lambda i: (i, 0)),
            pl.BlockSpec((HIDDEN, NUM_CLASSES), lambda i: (0, 0)),
            pl.BlockSpec((1, NUM_CLASSES), lambda i: (0, 0)),
        ],
        out_specs=pl.BlockSpec((m_blk, NUM_CLASSES), lambda i: (i, 0)),
        out_shape=jax.ShapeDtypeStruct((N, NUM_CLASSES), jnp.float32),
    )(pooled, w, b.reshape(1, NUM_CLASSES))


def _sc_body(h0_hbm, src_hbm, dst_hbm, w_hbm, pai_hbm, out_hbm,
             h_acc, h_bf, src_v, dst_v, w_v, rows_ba, rows_bb,
             rows_fa, rows_fb, zstage_v, pooled_v, coef_v,
             gsem_a, gsem_b, ssem_a, ssem_b):
    cid = lax.axis_index("c")
    sid = lax.axis_index("s")
    row0 = sid * RPT

    pltpu.sync_copy(src_hbm.at[sid], src_v)
    pltpu.sync_copy(dst_hbm.at[sid], dst_v)
    pltpu.sync_copy(w_hbm.at[sid], w_v)

    pltpu.sync_copy(pai_hbm, coef_v)
    coefs = jnp.exp(coef_v[...])
    c0 = coefs[0]

    def _zrow(r, _):
        for v in range(HC // 16):
            zstage_v[r, pl.ds(v * 16, 16)] = jnp.zeros((16,), jnp.float32)
        return 0
    lax.fori_loop(0, RSTEP, _zrow, 0)

    for j in range(RPT // RSTEP):
        r0 = row0 + j * RSTEP
        stage = rows_fa.at[pl.ds(0, RSTEP)]
        pltpu.sync_copy(h0_hbm.at[cid, pl.ds(r0, RSTEP)], stage)

        def _pinit(r, _, j=j):
            lo = rows_fa[r, pl.ds(0, 16)]
            hi = rows_fa[r, pl.ds(16, 16)]
            pooled_v[j * RSTEP + r, pl.ds(0, 16)] = c0 * lo
            pooled_v[j * RSTEP + r, pl.ds(16, 16)] = c0 * hi
            rows_ba[r, :] = plsc.pack(lo, hi, format=plsc.PackFormat.INTERLEAVED)
            return 0
        lax.fori_loop(0, RSTEP, _pinit, 0)
        pltpu.sync_copy(rows_ba.at[pl.ds(0, RSTEP)], h_bf.at[pl.ds(r0, RSTEP)])
        pltpu.sync_copy(zstage_v, h_acc.at[pl.ds(r0, RSTEP)])
    plsc.subcore_barrier()

    def _scale(bbuf, fbuf, k):
        def _group(g, _):
            wvec = w_v[k, pl.ds(g * 16, 16)]
            for e16 in range(16):
                e = g * 16 + e16
                w = wvec[e16]
                lo, hi = plsc.unpack(bbuf[e, :],
                                     format=plsc.PackFormat.INTERLEAVED)
                fbuf[e, pl.ds(0, 16)] = lo * w
                fbuf[e, pl.ds(16, 16)] = hi * w
            return 0
        lax.fori_loop(0, CHUNK // 16, _group, 0)

    for l in range(NUM_LAYERS):
        def _pair(k2, _):
            k0 = k2 * 2
            k1 = k0 + 1
            ga = pltpu.async_copy(h_bf.at[src_v.at[k0]], rows_ba, gsem_a)
            gb = pltpu.async_copy(h_bf.at[src_v.at[k1]], rows_bb, gsem_b)
            ga.wait()
            _scale(rows_ba, rows_fa, k0)
            sa = pltpu.async_copy(rows_fa, h_acc.at[dst_v.at[k0]], ssem_a,
                                  add=True)
            gb.wait()
            _scale(rows_bb, rows_fb, k1)
            sb = pltpu.async_copy(rows_fb, h_acc.at[dst_v.at[k1]], ssem_b,
                                  add=True)
            sa.wait()
            sb.wait()
            return 0
        lax.fori_loop(0, NCHUNK // 2, _pair, 0)
        plsc.subcore_barrier()

        cl = coefs[l + 1]
        for j in range(RPT // RSTEP):
            r0 = row0 + j * RSTEP
            stage = rows_fa.at[pl.ds(0, RSTEP)]
            pltpu.sync_copy(h_acc.at[pl.ds(r0, RSTEP)], stage)

            def _pacc(r, _, j=j):
                lo = rows_fa[r, pl.ds(0, 16)]
                hi = rows_fa[r, pl.ds(16, 16)]
                pooled_v[j * RSTEP + r, pl.ds(0, 16)] = (
                    pooled_v[j * RSTEP + r, pl.ds(0, 16)] + cl * lo)
                pooled_v[j * RSTEP + r, pl.ds(16, 16)] = (
                    pooled_v[j * RSTEP + r, pl.ds(16, 16)] + cl * hi)
                rows_ba[r, :] = plsc.pack(lo, hi,
                                          format=plsc.PackFormat.INTERLEAVED)
                return 0
            lax.fori_loop(0, RSTEP, _pacc, 0)
            if l < NUM_LAYERS - 1:
                pltpu.sync_copy(rows_ba.at[pl.ds(0, RSTEP)],
                                h_bf.at[pl.ds(r0, RSTEP)])
                pltpu.sync_copy(zstage_v, h_acc.at[pl.ds(r0, RSTEP)])
        plsc.subcore_barrier()

    pltpu.sync_copy(pooled_v, out_hbm.at[cid, pl.ds(row0, RPT)])


@functools.partial(
    pl.kernel,
    out_type=jax.ShapeDtypeStruct((NC, NP, HC), jnp.float32),
    mesh=plsc.VectorSubcoreMesh(core_axis_name="c", subcore_axis_name="s",
                                num_cores=NC, num_subcores=NS),
    scratch_types=[
        pltpu.VMEM_SHARED((NP, HC), jnp.float32),
        pltpu.VMEM_SHARED((NP, HC), jnp.bfloat16),
        pltpu.VMEM((NCHUNK, CHUNK), jnp.int32),
        pltpu.VMEM((NCHUNK, CHUNK), jnp.int32),
        pltpu.VMEM((NCHUNK, CHUNK), jnp.float32),
        pltpu.VMEM((CHUNK, HC), jnp.bfloat16),
        pltpu.VMEM((CHUNK, HC), jnp.bfloat16),
        pltpu.VMEM((CHUNK, HC), jnp.float32),
        pltpu.VMEM((CHUNK, HC), jnp.float32),
        pltpu.VMEM((RSTEP, HC), jnp.float32),
        pltpu.VMEM((RPT, HC), jnp.float32),
        pltpu.VMEM((16,), jnp.float32),
        pltpu.SemaphoreType.DMA,
        pltpu.SemaphoreType.DMA,
        pltpu.SemaphoreType.DMA,
        pltpu.SemaphoreType.DMA,
    ],
    compiler_params=pltpu.CompilerParams(use_tc_tiling_on_sc=False,
                                         needs_layout_passes=False),
)
def _sc_propagate(h0_hbm, src_hbm, dst_hbm, w_hbm, pai_hbm, out_hbm,
                  h_acc, h_bf, src_v, dst_v, w_v, rows_ba, rows_bb,
                  rows_fa, rows_fb, zstage_v, pooled_v, coef_v,
                  gsem_a, gsem_b, ssem_a, ssem_b):
    _sc_body(h0_hbm, src_hbm, dst_hbm, w_hbm, pai_hbm, out_hbm,
             h_acc, h_bf, src_v, dst_v, w_v, rows_ba, rows_bb,
             rows_fa, rows_fb, zstage_v, pooled_v, coef_v,
             gsem_a, gsem_b, ssem_a, ssem_b)


def _pad_edges(a):
    a = a.reshape(NS, E // NS)
    a = jnp.pad(a, ((0, 0), (0, EPT - E // NS)))
    return a.reshape(NS, NCHUNK, CHUNK)


def kernel(x, edge_index, edge_weight, mask, type, W0, b0, W1, b1, pai, convW):
    del mask, type, convW
    src = _pad_edges(edge_index[0].astype(jnp.int32))
    dst = _pad_edges(edge_index[1].astype(jnp.int32))
    w = _pad_edges(edge_weight.astype(jnp.float32))
    pai_pad = jnp.pad(pai.reshape(-1), (0, 16 - (NUM_LAYERS + 1)))

    h0 = _mm_relu(x, W0, b0)
    h0_split = h0.reshape(NP, NC, HC).transpose(1, 0, 2)
    pooled_split = _sc_propagate(h0_split, src, dst, w, pai_pad)
    pooled = pooled_split.transpose(1, 0, 2).reshape(NP, HIDDEN)
    return _head(pooled, W1, b1)


# speedup vs baseline: 1.7182x; 1.5096x over previous
import functools

import jax
import jax.numpy as jnp
from jax import lax
from jax.experimental import pallas as pl
from jax.experimental.pallas import tpu as pltpu
from jax.experimental.pallas import tpu_sc as plsc

N = 10000
E = 320000
D_FEAT = 128
HIDDEN = 64
NUM_CLASSES = 40
NUM_LAYERS = 8

NC = 2
NS = 16
HC = HIDDEN // NC
NP = N
CHUNK = 128
EPT = 20480
NCHUNK = EPT // CHUNK
RPT = NP // NS
RSTEP = 125


def _mm_relu_body(x_ref, w_ref, b_ref, o_ref):
    acc = jnp.dot(x_ref[...], w_ref[...], preferred_element_type=jnp.float32)
    o_ref[...] = jnp.maximum(acc + b_ref[...], 0.0)


def _mm_relu(x, w, b):
    m_blk = 2000
    grid = (N // m_blk,)
    return pl.pallas_call(
        _mm_relu_body,
        grid=grid,
        in_specs=[
            pl.BlockSpec((m_blk, D_FEAT), lambda i: (i, 0)),
            pl.BlockSpec((D_FEAT, HIDDEN), lambda i: (0, 0)),
            pl.BlockSpec((1, HIDDEN), lambda i: (0, 0)),
        ],
        out_specs=pl.BlockSpec((m_blk, HIDDEN), lambda i: (i, 0)),
        out_shape=jax.ShapeDtypeStruct((N, HIDDEN), jnp.float32),
    )(x, w, b.reshape(1, HIDDEN))


def _head_body(p_ref, w_ref, b_ref, o_ref):
    logits = jnp.dot(p_ref[...], w_ref[...], preferred_element_type=jnp.float32)
    logits = logits + b_ref[...]
    mx = jnp.max(logits, axis=-1, keepdims=True)
    z = logits - mx
    lse = jnp.log(jnp.sum(jnp.exp(z), axis=-1, keepdims=True))
    o_ref[...] = z - lse


def _head(pooled, w, b):
    m_blk = 2000
    grid = (N // m_blk,)
    return pl.pallas_call(
        _head_body,
        grid=grid,
        in_specs=[
            pl.BlockSpec((m_blk, HIDDEN), lambda i: (i, 0)),
            pl.BlockSpec((HIDDEN, NUM_CLASSES), lambda i: (0, 0)),
            pl.BlockSpec((1, NUM_CLASSES), lambda i: (0, 0)),
        ],
        out_specs=pl.BlockSpec((m_blk, NUM_CLASSES), lambda i: (i, 0)),
        out_shape=jax.ShapeDtypeStruct((N, NUM_CLASSES), jnp.float32),
    )(pooled, w, b.reshape(1, NUM_CLASSES))


def _sc_body(h0_hbm, src_hbm, dst_hbm, w_hbm, pai_hbm, out_hbm,
             h_a, h_b, src_v, dst_v, w_v, rows_va, rows_vb, pooled_v, coef_v,
             gsem_a, gsem_b, ssem_a, ssem_b):
    cid = lax.axis_index("c")
    sid = lax.axis_index("s")
    row0 = sid * RPT

    pltpu.sync_copy(src_hbm.at[sid], src_v)
    pltpu.sync_copy(dst_hbm.at[sid], dst_v)
    pltpu.sync_copy(w_hbm.at[sid], w_v)

    pltpu.sync_copy(pai_hbm, coef_v)
    coefs = jnp.exp(coef_v[...])
    c0 = coefs[0]

    for j in range(RPT // RSTEP):
        r0 = row0 + j * RSTEP
        stage = rows_va.at[pl.ds(0, RSTEP)]
        pltpu.sync_copy(h0_hbm.at[cid, pl.ds(r0, RSTEP)], stage)
        pltpu.sync_copy(stage, h_a.at[pl.ds(r0, RSTEP)])

        def _pinit(r, _, j=j):
            for v in range(HC // 16):
                sl = pl.ds(v * 16, 16)
                pooled_v[j * RSTEP + r, sl] = c0 * rows_va[r, sl]
            return 0
        lax.fori_loop(0, RSTEP, _pinit, 0)

    def _scale(buf, k):
        def _group(g, _):
            wvec = w_v[k, pl.ds(g * 16, 16)]
            for e16 in range(16):
                e = g * 16 + e16
                w = wvec[e16]
                for v in range(HC // 16):
                    sl = pl.ds(v * 16, 16)
                    buf[e, sl] = buf[e, sl] * w
            return 0
        lax.fori_loop(0, CHUNK // 16, _group, 0)

    for l in range(NUM_LAYERS):
        h_in, h_out = (h_a, h_b) if l % 2 == 0 else (h_b, h_a)

        def _zrow(r, _):
            for v in range(HC // 16):
                rows_va[r, pl.ds(v * 16, 16)] = jnp.zeros((16,), jnp.float32)
            return 0
        lax.fori_loop(0, RSTEP, _zrow, 0)
        zcps = [
            pltpu.async_copy(rows_va.at[pl.ds(0, RSTEP)],
                             h_out.at[pl.ds(row0 + j * RSTEP, RSTEP)],
                             gsem_a)
            for j in range(RPT // RSTEP)
        ]
        for cp in zcps:
            cp.wait()
        plsc.subcore_barrier()

        def _pair(k2, _):
            k0 = k2 * 2
            k1 = k0 + 1
            ga = pltpu.async_copy(h_in.at[src_v.at[k0]], rows_va, gsem_a)
            gb = pltpu.async_copy(h_in.at[src_v.at[k1]], rows_vb, gsem_b)
            ga.wait()
            _scale(rows_va, k0)
            sa = pltpu.async_copy(rows_va, h_out.at[dst_v.at[k0]], ssem_a,
                                  add=True)
            gb.wait()
            _scale(rows_vb, k1)
            sb = pltpu.async_copy(rows_vb, h_out.at[dst_v.at[k1]], ssem_b,
                                  add=True)
            sa.wait()
            sb.wait()
            return 0
        lax.fori_loop(0, NCHUNK // 2, _pair, 0)
        plsc.subcore_barrier()

        cl = coefs[l + 1]
        nj = RPT // RSTEP
        bufs = [rows_va if j % 2 == 0 else rows_vb for j in range(nj)]
        sems = [gsem_a if j % 2 == 0 else gsem_b for j in range(nj)]
        cps = [None] * nj
        cps[0] = pltpu.async_copy(
            h_out.at[pl.ds(row0, RSTEP)], bufs[0].at[pl.ds(0, RSTEP)],
            sems[0])
        for j in range(nj):
            cps[j].wait()
            if j + 1 < nj:
                cps[j + 1] = pltpu.async_copy(
                    h_out.at[pl.ds(row0 + (j + 1) * RSTEP, RSTEP)],
                    bufs[j + 1].at[pl.ds(0, RSTEP)], sems[j + 1])
            buf = bufs[j]

            def _pacc(r, _, j=j, buf=buf):
                for v in range(HC // 16):
                    sl = pl.ds(v * 16, 16)
                    pooled_v[j * RSTEP + r, sl] = (
                        pooled_v[j * RSTEP + r, sl] + cl * buf[r, sl])
                return 0
            lax.fori_loop(0, RSTEP, _pacc, 0)

    pltpu.sync_copy(pooled_v, out_hbm.at[cid, pl.ds(row0, RPT)])


@functools.partial(
    pl.kernel,
    out_type=jax.ShapeDtypeStruct((NC, NP, HC), jnp.float32),
    mesh=plsc.VectorSubcoreMesh(core_axis_name="c", subcore_axis_name="s",
                                num_cores=NC, num_subcores=NS),
    scratch_types=[
        pltpu.VMEM_SHARED((NP, HC), jnp.float32),
        pltpu.VMEM_SHARED((NP, HC), jnp.float32),
        pltpu.VMEM((NCHUNK, CHUNK), jnp.int32),
        pltpu.VMEM((NCHUNK, CHUNK), jnp.int32),
        pltpu.VMEM((NCHUNK, CHUNK), jnp.float32),
        pltpu.VMEM((CHUNK, HC), jnp.float32),
        pltpu.VMEM((CHUNK, HC), jnp.float32),
        pltpu.VMEM((RPT, HC), jnp.float32),
        pltpu.VMEM((16,), jnp.float32),
        pltpu.SemaphoreType.DMA,
        pltpu.SemaphoreType.DMA,
        pltpu.SemaphoreType.DMA,
        pltpu.SemaphoreType.DMA,
    ],
    compiler_params=pltpu.CompilerParams(use_tc_tiling_on_sc=False),
)
def _sc_propagate(h0_hbm, src_hbm, dst_hbm, w_hbm, pai_hbm, out_hbm,
                  h_a, h_b, src_v, dst_v, w_v, rows_va, rows_vb, pooled_v,
                  coef_v, gsem_a, gsem_b, ssem_a, ssem_b):
    _sc_body(h0_hbm, src_hbm, dst_hbm, w_hbm, pai_hbm, out_hbm,
             h_a, h_b, src_v, dst_v, w_v, rows_va, rows_vb, pooled_v,
             coef_v, gsem_a, gsem_b, ssem_a, ssem_b)


def _pad_edges(a):
    a = a.reshape(NS, E // NS)
    a = jnp.pad(a, ((0, 0), (0, EPT - E // NS)))
    return a.reshape(NS, NCHUNK, CHUNK)


def kernel(x, edge_index, edge_weight, mask, type, W0, b0, W1, b1, pai, convW):
    del mask, type, convW
    src = _pad_edges(edge_index[0].astype(jnp.int32))
    dst = _pad_edges(edge_index[1].astype(jnp.int32))
    w = _pad_edges(edge_weight.astype(jnp.float32))
    pai_pad = jnp.pad(pai.reshape(-1), (0, 16 - (NUM_LAYERS + 1)))

    h0 = _mm_relu(x, W0, b0)
    h0_split = h0.reshape(NP, NC, HC).transpose(1, 0, 2)
    pooled_split = _sc_propagate(h0_split, src, dst, w, pai_pad)
    pooled = pooled_split.transpose(1, 0, 2).reshape(NP, HIDDEN)
    return _head(pooled, W1, b1)


# speedup vs baseline: 1.7438x; 1.0149x over previous
import functools

import jax
import jax.numpy as jnp
from jax import lax
from jax.experimental import pallas as pl
from jax.experimental.pallas import tpu as pltpu
from jax.experimental.pallas import tpu_sc as plsc

N = 10000
E = 320000
D_FEAT = 128
HIDDEN = 64
NUM_CLASSES = 40
NUM_LAYERS = 8

NC = 2
NS = 16
HC = HIDDEN // NC
NP = N
CHUNK = 128
EPT = 20480
NCHUNK = EPT // CHUNK
RPT = NP // NS
RSTEP = 125


def _mm_relu_body(x_ref, w_ref, b_ref, o_ref):
    acc = jnp.dot(x_ref[...], w_ref[...], preferred_element_type=jnp.float32)
    acc = jnp.maximum(acc + b_ref[...], 0.0)
    m_blk = acc.shape[0]
    o_ref[...] = acc.reshape(m_blk, NC, HC).transpose(1, 0, 2)


def _mm_relu(x, w, b):
    m_blk = 2000
    grid = (N // m_blk,)
    return pl.pallas_call(
        _mm_relu_body,
        grid=grid,
        in_specs=[
            pl.BlockSpec((m_blk, D_FEAT), lambda i: (i, 0)),
            pl.BlockSpec((D_FEAT, HIDDEN), lambda i: (0, 0)),
            pl.BlockSpec((1, HIDDEN), lambda i: (0, 0)),
        ],
        out_specs=pl.BlockSpec((NC, m_blk, HC), lambda i: (0, i, 0)),
        out_shape=jax.ShapeDtypeStruct((NC, N, HC), jnp.float32),
    )(x, w, b.reshape(1, HIDDEN))


def _head_body(p_ref, w_ref, b_ref, o_ref):
    m_blk = p_ref.shape[1]
    pooled = p_ref[...].transpose(1, 0, 2).reshape(m_blk, HIDDEN)
    logits = jnp.dot(pooled, w_ref[...], preferred_element_type=jnp.float32)
    logits = logits + b_ref[...]
    mx = jnp.max(logits, axis=-1, keepdims=True)
    z = logits - mx
    lse = jnp.log(jnp.sum(jnp.exp(z), axis=-1, keepdims=True))
    o_ref[...] = z - lse


def _head(pooled, w, b):
    m_blk = 2000
    grid = (N // m_blk,)
    return pl.pallas_call(
        _head_body,
        grid=grid,
        in_specs=[
            pl.BlockSpec((NC, m_blk, HC), lambda i: (0, i, 0)),
            pl.BlockSpec((HIDDEN, NUM_CLASSES), lambda i: (0, 0)),
            pl.BlockSpec((1, NUM_CLASSES), lambda i: (0, 0)),
        ],
        out_specs=pl.BlockSpec((m_blk, NUM_CLASSES), lambda i: (i, 0)),
        out_shape=jax.ShapeDtypeStruct((N, NUM_CLASSES), jnp.float32),
    )(pooled, w, b.reshape(1, NUM_CLASSES))


def _sc_body(h0_hbm, src_hbm, dst_hbm, w_hbm, pai_hbm, out_hbm,
             h_a, h_b, src_v, dst_v, w_v, rows_va, rows_vb, pooled_v, coef_v,
             gsem_a, gsem_b, ssem_a, ssem_b):
    cid = lax.axis_index("c")
    sid = lax.axis_index("s")
    row0 = sid * RPT

    pltpu.sync_copy(src_hbm.at[sid], src_v)
    pltpu.sync_copy(dst_hbm.at[sid], dst_v)
    pltpu.sync_copy(w_hbm.at[sid], w_v)

    pltpu.sync_copy(pai_hbm, coef_v)
    coefs = jnp.exp(coef_v[...])
    c0 = coefs[0]

    for j in range(RPT // RSTEP):
        r0 = row0 + j * RSTEP
        stage = rows_va.at[pl.ds(0, RSTEP)]
        pltpu.sync_copy(h0_hbm.at[cid, pl.ds(r0, RSTEP)], stage)
        pltpu.sync_copy(stage, h_a.at[pl.ds(r0, RSTEP)])

        def _pinit(r, _, j=j):
            for v in range(HC // 16):
                sl = pl.ds(v * 16, 16)
                pooled_v[j * RSTEP + r, sl] = c0 * rows_va[r, sl]
            return 0
        lax.fori_loop(0, RSTEP, _pinit, 0)

    def _scale(buf, k):
        def _group(g, _):
            wvec = w_v[k, pl.ds(g * 16, 16)]
            for e16 in range(16):
                e = g * 16 + e16
                w = wvec[e16]
                for v in range(HC // 16):
                    sl = pl.ds(v * 16, 16)
                    buf[e, sl] = buf[e, sl] * w
            return 0
        lax.fori_loop(0, CHUNK // 16, _group, 0)

    for l in range(NUM_LAYERS):
        h_in, h_out = (h_a, h_b) if l % 2 == 0 else (h_b, h_a)

        def _zrow(r, _):
            for v in range(HC // 16):
                rows_va[r, pl.ds(v * 16, 16)] = jnp.zeros((16,), jnp.float32)
            return 0
        lax.fori_loop(0, RSTEP, _zrow, 0)
        zcps = [
            pltpu.async_copy(rows_va.at[pl.ds(0, RSTEP)],
                             h_out.at[pl.ds(row0 + j * RSTEP, RSTEP)],
                             gsem_a)
            for j in range(RPT // RSTEP)
        ]
        for cp in zcps:
            cp.wait()
        plsc.subcore_barrier()

        def _pair(k2, _):
            k0 = k2 * 2
            k1 = k0 + 1
            ga = pltpu.async_copy(h_in.at[src_v.at[k0]], rows_va, gsem_a)
            gb = pltpu.async_copy(h_in.at[src_v.at[k1]], rows_vb, gsem_b)
            ga.wait()
            _scale(rows_va, k0)
            sa = pltpu.async_copy(rows_va, h_out.at[dst_v.at[k0]], ssem_a,
                                  add=True)
            gb.wait()
            _scale(rows_vb, k1)
            sb = pltpu.async_copy(rows_vb, h_out.at[dst_v.at[k1]], ssem_b,
                                  add=True)
            sa.wait()
            sb.wait()
            return 0
        lax.fori_loop(0, NCHUNK // 2, _pair, 0)
        plsc.subcore_barrier()

        cl = coefs[l + 1]
        nj = RPT // RSTEP
        bufs = [rows_va if j % 2 == 0 else rows_vb for j in range(nj)]
        sems = [gsem_a if j % 2 == 0 else gsem_b for j in range(nj)]
        cps = [None] * nj
        cps[0] = pltpu.async_copy(
            h_out.at[pl.ds(row0, RSTEP)], bufs[0].at[pl.ds(0, RSTEP)],
            sems[0])
        for j in range(nj):
            cps[j].wait()
            if j + 1 < nj:
                cps[j + 1] = pltpu.async_copy(
                    h_out.at[pl.ds(row0 + (j + 1) * RSTEP, RSTEP)],
                    bufs[j + 1].at[pl.ds(0, RSTEP)], sems[j + 1])
            buf = bufs[j]

            def _pacc(r, _, j=j, buf=buf):
                for v in range(HC // 16):
                    sl = pl.ds(v * 16, 16)
                    pooled_v[j * RSTEP + r, sl] = (
                        pooled_v[j * RSTEP + r, sl] + cl * buf[r, sl])
                return 0
            lax.fori_loop(0, RSTEP, _pacc, 0)

    pltpu.sync_copy(pooled_v, out_hbm.at[cid, pl.ds(row0, RPT)])


@functools.partial(
    pl.kernel,
    out_type=jax.ShapeDtypeStruct((NC, NP, HC), jnp.float32),
    mesh=plsc.VectorSubcoreMesh(core_axis_name="c", subcore_axis_name="s",
                                num_cores=NC, num_subcores=NS),
    scratch_types=[
        pltpu.VMEM_SHARED((NP, HC), jnp.float32),
        pltpu.VMEM_SHARED((NP, HC), jnp.float32),
        pltpu.VMEM((NCHUNK, CHUNK), jnp.int32),
        pltpu.VMEM((NCHUNK, CHUNK), jnp.int32),
        pltpu.VMEM((NCHUNK, CHUNK), jnp.float32),
        pltpu.VMEM((CHUNK, HC), jnp.float32),
        pltpu.VMEM((CHUNK, HC), jnp.float32),
        pltpu.VMEM((RPT, HC), jnp.float32),
        pltpu.VMEM((16,), jnp.float32),
        pltpu.SemaphoreType.DMA,
        pltpu.SemaphoreType.DMA,
        pltpu.SemaphoreType.DMA,
        pltpu.SemaphoreType.DMA,
    ],
    compiler_params=pltpu.CompilerParams(use_tc_tiling_on_sc=False),
)
def _sc_propagate(h0_hbm, src_hbm, dst_hbm, w_hbm, pai_hbm, out_hbm,
                  h_a, h_b, src_v, dst_v, w_v, rows_va, rows_vb, pooled_v,
                  coef_v, gsem_a, gsem_b, ssem_a, ssem_b):
    _sc_body(h0_hbm, src_hbm, dst_hbm, w_hbm, pai_hbm, out_hbm,
             h_a, h_b, src_v, dst_v, w_v, rows_va, rows_vb, pooled_v,
             coef_v, gsem_a, gsem_b, ssem_a, ssem_b)


def _pad_edges(a):
    a = a.reshape(NS, E // NS)
    a = jnp.pad(a, ((0, 0), (0, EPT - E // NS)))
    return a.reshape(NS, NCHUNK, CHUNK)


def kernel(x, edge_index, edge_weight, mask, type, W0, b0, W1, b1, pai, convW):
    del mask, type, convW
    src = _pad_edges(edge_index[0].astype(jnp.int32))
    dst = _pad_edges(edge_index[1].astype(jnp.int32))
    w = _pad_edges(edge_weight.astype(jnp.float32))
    pai_pad = jnp.pad(pai.reshape(-1), (0, 16 - (NUM_LAYERS + 1)))

    h0_split = _mm_relu(x, W0, b0)
    pooled_split = _sc_propagate(h0_split, src, dst, w, pai_pad)
    return _head(pooled_split, W1, b1)
